# baseline (speedup 1.0000x reference)
import functools
import jax
import jax.numpy as jnp
from jax import lax
from jax.experimental import pallas as pl
from jax.experimental.pallas import tpu as pltpu
from jax.experimental.pallas import tpu_sc as plsc


def _make_sc_copy():
    mesh = plsc.VectorSubcoreMesh(core_axis_name="c", subcore_axis_name="s")

    @functools.partial(
        pl.kernel, mesh=mesh,
        out_type=jax.ShapeDtypeStruct((16, 128), jnp.float32),
        scratch_types=[pltpu.VMEM((16, 128), jnp.float32)],
    )
    def k(x_hbm, out_hbm, buf):
        wid = lax.axis_index("s") * 2 + lax.axis_index("c")

        @pl.when(wid == 0)
        def _():
            pltpu.sync_copy(x_hbm, buf)
            pltpu.sync_copy(buf, out_hbm)

    return k


def kernel(sp_x, sp_edge_index, params):
    xp = jnp.zeros((16, 128), jnp.float32).at[:, :45].set(sp_x)
    y = _make_sc_copy()(xp)
    return y[0, :].repeat(2)[:256]


# speedup vs baseline: 1.7854x; 1.7854x over previous
import jax
import jax.numpy as jnp
from jax.experimental import pallas as pl
from jax.experimental.pallas import tpu as pltpu

N = 16
E = 64
H = 128

_RS = 40


def _net_kernel(x_ref, ei_ref, sp_ref,
                w0_ref, w1_ref, w2_ref, w3_ref, lin_ref,
                f0_ref, f1_ref, f2_ref, f3_ref, out_ref):
    f32 = jnp.float32
    w_refs = (w0_ref, w1_ref, w2_ref, w3_ref)
    f_refs = (f0_ref, f1_ref, f2_ref, f3_ref)

    def dotT(a, b):
        return jax.lax.dot_general(a, b, (((0,), (0,)), ((), ())),
                                   preferred_element_type=f32)

    def mm(a, b):
        return jax.lax.dot_general(a, b, (((1,), (0,)), ((), ())),
                                   preferred_element_type=f32)

    def rowdot(a, w_row):
        return jnp.sum(a * w_row, axis=1, keepdims=True)

    sp = sp_ref[:, :]

    srcT = ei_ref[0:1, :]
    dstT = ei_ref[1:2, :]
    rowN = jax.lax.broadcasted_iota(jnp.int32, (N, E), 0)
    ST = (srcT == rowN).astype(f32)
    DT = (dstT == rowN).astype(f32)
    mask = jnp.ones((1, E), dtype=f32)

    row_i = jax.lax.broadcasted_iota(jnp.int32, (N, N), 0)
    col_i = jax.lax.broadcasted_iota(jnp.int32, (N, N), 1)
    eye = (row_i == col_i).astype(f32)
    colf = col_i.astype(f32)
    valid_col = jax.lax.broadcasted_iota(jnp.int32, (N, 1), 0)

    S = dotT(ST, eye)
    x = x_ref[:, :]

    n_cur = N
    for l in range(4):
        W = w_refs[l][:, :]
        b = sp[l:l + 1, :]
        wrel = sp[4 + l:5 + l, :]
        wroot = sp[8 + l:9 + l, :]
        brel = sp[33:34, l:l + 1]

        DTm = DT * mask
        deg = jnp.sum(DTm, axis=1, keepdims=True) + 1.0
        dinv = 1.0 / jnp.sqrt(deg)
        norm = mask * dotT(dinv, ST) * dotT(dinv, DT)
        M = mm(DT * norm, S) + eye * (1.0 / deg)
        xw = mm(x, W)
        x = jax.nn.relu(mm(M, xw) + b)

        B = mm(DTm, S)
        raw = mm(B, rowdot(x, wrel)) + rowdot(x, wroot) + brel
        score = jnp.tanh(raw)
        score = jnp.where(valid_col < n_cur, score, -2.0)

        k = (n_cur + 1) // 2
        s_row = dotT(score, eye)
        s_cb = jax.lax.broadcast_in_dim(score, (N, N), (0, 1))
        s_rb = jax.lax.broadcast_in_dim(s_row, (N, N), (0, 1))
        beats = (s_rb > s_cb) | ((s_rb == s_cb) & (col_i < row_i))
        rank = jnp.sum(beats.astype(f32), axis=1, keepdims=True)
        PT = ((rank == colf) & (colf < float(k))).astype(f32)

        x = dotT(PT * score, x)
        S = mm(S, PT)
        ST = dotT(PT, ST)
        DT = dotT(PT, DT)
        mask = (mask * jnp.sum(ST, axis=0, keepdims=True)
                     * jnp.sum(DT, axis=0, keepdims=True))
        n_cur = k

    out2 = mm(x[0:1, :], lin_ref[:, :]) + sp[12:13, :]
    m = jnp.max(out2, axis=1, keepdims=True)
    z = out2 - m
    out2 = z - jnp.log(jnp.sum(jnp.exp(z), axis=1, keepdims=True))

    h = jnp.concatenate([jnp.zeros((1, H), dtype=f32), out2], axis=1)

    def row256(r):
        return jnp.concatenate([sp[r:r + 1, :], sp[r + 1:r + 2, :]], axis=1)

    for l in range(4):
        h = mm(h, f_refs[l][:, :])
        if l < 3:
            h = h + row256(13 + 6 * l)
            mu = jnp.mean(h, axis=1, keepdims=True)
            var = jnp.mean((h - mu) ** 2, axis=1, keepdims=True)
            h = ((h - mu) / jnp.sqrt(var + 1e-5) * row256(15 + 6 * l)
                 + row256(17 + 6 * l))
            h = jax.nn.relu(h)
        else:
            h = h + row256(31)

    out_ref[:, :] = h


def kernel(sp_x, sp_edge_index, params):
    f32 = jnp.float32
    p = params

    parts = []
    for l in range(4):
        parts.append(p['conv%d_b' % l].reshape(-1))
    for l in range(4):
        parts.append(p['pool%d_Wrel' % l].reshape(-1))
    for l in range(4):
        parts.append(p['pool%d_Wroot' % l].reshape(-1))
    parts.append(p['lin_b'].reshape(-1))
    for l in range(3):
        parts += [p['fc%d_b' % l].reshape(-1),
                  p['ln%d_w' % l].reshape(-1),
                  p['ln%d_b' % l].reshape(-1)]
    parts.append(p['fc3_b'].reshape(-1))
    for l in range(4):
        parts.append(p['pool%d_brel' % l])
    parts.append(jnp.zeros((124 + 6 * H,), f32))
    smallpack = jnp.concatenate(parts).reshape(_RS, H)

    out = pl.pallas_call(
        _net_kernel,
        out_shape=jax.ShapeDtypeStruct((1, 256), f32),
    )(sp_x, sp_edge_index, smallpack,
      p['conv0_W'], p['conv1_W'], p['conv2_W'], p['conv3_W'], p['lin_W'],
      p['fc0_W'], p['fc1_W'], p['fc2_W'], p['fc3_W'])
    return out.reshape(-1)
